# baseline (speedup 1.0000x reference)
import functools

import jax
import jax.numpy as jnp
from jax import lax
from jax.experimental import pallas as pl
from jax.experimental.pallas import tpu as pltpu
from jax.experimental.pallas import tpu_sc as plsc

NC = 2
NS = 16
NW = NC * NS
NBUF = 8


@functools.partial(jax.jit, static_argnames=("b", "h", "hp", "d"))
def _sc_gather(idx2_flat, pe, b, h, hp, d):
    batches_per_w = b // NW
    pairs_per_w = batches_per_w // 2
    rounds = pairs_per_w // NBUF
    idx_per_w = batches_per_w * hp
    mesh = plsc.VectorSubcoreMesh(core_axis_name="c", subcore_axis_name="s")

    @functools.partial(
        pl.kernel,
        mesh=mesh,
        out_type=jax.ShapeDtypeStruct((b, h, d), jnp.float32),
        scratch_types=[
            pltpu.VMEM((idx_per_w,), jnp.int32),
            pltpu.VMEM((NBUF, 2, h, d), jnp.float32),
        ]
        + [pltpu.SemaphoreType.DMA] * (2 * NBUF),
    )
    def k(idx_hbm, table_hbm, out_hbm, idx_v, rows_v, *sems):
        gsem = sems[:NBUF]
        wsem = sems[NBUF:]
        wid = lax.axis_index("s") * NC + lax.axis_index("c")
        batch0 = wid * batches_per_w

        pltpu.sync_copy(idx_hbm.at[pl.ds(wid * idx_per_w, idx_per_w)], idx_v)

        def start_gather(j, bf):
            for t in range(2):
                pltpu.async_copy(
                    table_hbm.at[idx_v.at[pl.ds((2 * j + t) * hp, h)]],
                    rows_v.at[bf].at[t],
                    gsem[bf],
                )

        def wait_gather(j, bf):
            for t in range(2):
                pltpu.make_async_copy(
                    table_hbm.at[idx_v.at[pl.ds((2 * j + t) * hp, h)]],
                    rows_v.at[bf].at[t],
                    gsem[bf],
                ).wait()

        def start_write(j, bf):
            pltpu.async_copy(
                rows_v.at[bf], out_hbm.at[pl.ds(batch0 + 2 * j, 2)], wsem[bf]
            )

        def wait_write(j, bf):
            pltpu.make_async_copy(
                rows_v.at[bf], out_hbm.at[pl.ds(batch0 + 2 * j, 2)], wsem[bf]
            ).wait()

        for bf in range(NBUF):
            start_gather(bf, bf)

        def body(r, carry):
            j0 = r * NBUF
            for bf in range(NBUF):
                wait_gather(j0 + bf, bf)
                start_write(j0 + bf, bf)
            for bf in range(NBUF):
                wait_write(j0 + bf, bf)
                start_gather(j0 + NBUF + bf, bf)
            return carry

        lax.fori_loop(0, rounds - 1, body, 0)

        j0 = (rounds - 1) * NBUF
        for bf in range(NBUF):
            wait_gather(j0 + bf, bf)
            start_write(j0 + bf, bf)
        for bf in range(NBUF):
            wait_write(j0 + bf, bf)

    return k(idx2_flat, pe)


def kernel(timestamps, pe):
    b, h = timestamps.shape
    d = pe.shape[1]
    hp = (h + 7) // 8 * 8
    idx2 = jnp.pad(timestamps, ((0, 0), (0, hp - h)), mode="edge")
    return _sc_gather(idx2.reshape(-1), pe, b, h, hp, d)


# speedup vs baseline: 1.0124x; 1.0124x over previous
import functools

import jax
import jax.numpy as jnp
from jax import lax
from jax.experimental import pallas as pl
from jax.experimental.pallas import tpu as pltpu
from jax.experimental.pallas import tpu_sc as plsc

NC = 2
NS = 16
NW = NC * NS
NBUF = 8


@functools.partial(jax.jit, static_argnames=("b", "h", "hp", "d"))
def _sc_gather(idx2_flat, pe, b, h, hp, d):
    batches_per_w = b // NW
    rounds = batches_per_w // NBUF
    idx_per_w = batches_per_w * hp
    mesh = plsc.VectorSubcoreMesh(core_axis_name="c", subcore_axis_name="s")

    @functools.partial(
        pl.kernel,
        mesh=mesh,
        out_type=jax.ShapeDtypeStruct((b, h, d), jnp.float32),
        scratch_types=[
            pltpu.VMEM((idx_per_w,), jnp.int32),
            pltpu.VMEM((NBUF, h, d), jnp.float32),
        ]
        + [pltpu.SemaphoreType.DMA] * (2 * NBUF),
    )
    def k(idx_hbm, table_hbm, out_hbm, idx_v, rows_v, *sems):
        gsem = sems[:NBUF]
        wsem = sems[NBUF:]
        wid = lax.axis_index("c") * NS + lax.axis_index("s")
        batch0 = wid * batches_per_w

        pltpu.sync_copy(idx_hbm.at[pl.ds(wid * idx_per_w, idx_per_w)], idx_v)

        def start_gather(j, bf):
            pltpu.async_copy(
                table_hbm.at[idx_v.at[pl.ds(j * hp, h)]], rows_v.at[bf], gsem[bf]
            )

        def wait_gather(j, bf):
            pltpu.make_async_copy(
                table_hbm.at[idx_v.at[pl.ds(j * hp, h)]], rows_v.at[bf], gsem[bf]
            ).wait()

        def start_write(j, bf):
            pltpu.async_copy(rows_v.at[bf], out_hbm.at[batch0 + j], wsem[bf])

        def wait_write(j, bf):
            pltpu.make_async_copy(
                rows_v.at[bf], out_hbm.at[batch0 + j], wsem[bf]
            ).wait()

        for bf in range(NBUF):
            start_gather(bf, bf)

        def body(r, carry):
            j0 = r * NBUF
            for bf in range(NBUF):
                wait_gather(j0 + bf, bf)
                start_write(j0 + bf, bf)
            for bf in range(NBUF):
                wait_write(j0 + bf, bf)
                start_gather(j0 + NBUF + bf, bf)
            return carry

        lax.fori_loop(0, rounds - 1, body, 0)

        j0 = (rounds - 1) * NBUF
        for bf in range(NBUF):
            wait_gather(j0 + bf, bf)
            start_write(j0 + bf, bf)
        for bf in range(NBUF):
            wait_write(j0 + bf, bf)

    return k(idx2_flat, pe)


def kernel(timestamps, pe):
    b, h = timestamps.shape
    d = pe.shape[1]
    hp = (h + 7) // 8 * 8
    idx2 = jnp.pad(timestamps, ((0, 0), (0, hp - h)), mode="edge")
    return _sc_gather(idx2.reshape(-1), pe, b, h, hp, d)


# speedup vs baseline: 1.0182x; 1.0057x over previous
import functools

import jax
import jax.numpy as jnp
from jax import lax
from jax.experimental import pallas as pl
from jax.experimental.pallas import tpu as pltpu
from jax.experimental.pallas import tpu_sc as plsc

NC = 2
NS = 16
NW = NC * NS
NBUF = 8


@functools.partial(jax.jit, static_argnames=("b", "h", "hp", "d"))
def _sc_gather(idx2_flat, pe, b, h, hp, d):
    batches_per_w = b // NW
    rounds = batches_per_w // NBUF
    idx_per_w = batches_per_w * hp
    mesh = plsc.VectorSubcoreMesh(core_axis_name="c", subcore_axis_name="s")

    @functools.partial(
        pl.kernel,
        mesh=mesh,
        out_type=jax.ShapeDtypeStruct((b, h, d), jnp.float32),
        scratch_types=[
            pltpu.VMEM((idx_per_w,), jnp.int32),
            pltpu.VMEM((NBUF, h, d), jnp.float32),
        ]
        + [pltpu.SemaphoreType.DMA] * (2 * NBUF),
    )
    def k(idx_hbm, table_hbm, out_hbm, idx_v, rows_v, *sems):
        gsem = sems[:NBUF]
        wsem = sems[NBUF:]
        wid = lax.axis_index("c") * NS + lax.axis_index("s")
        batch0 = wid * batches_per_w

        pltpu.sync_copy(idx_hbm.at[pl.ds(wid * idx_per_w, idx_per_w)], idx_v)

        def start_gather(j, bf):
            pltpu.async_copy(
                table_hbm.at[idx_v.at[pl.ds(j * hp, h)]], rows_v.at[bf], gsem[bf]
            )

        def wait_gather(j, bf):
            pltpu.make_async_copy(
                table_hbm.at[idx_v.at[pl.ds(j * hp, h)]], rows_v.at[bf], gsem[bf]
            ).wait()

        def start_write(j, bf):
            pltpu.async_copy(rows_v.at[bf], out_hbm.at[batch0 + j], wsem[bf])

        def wait_write(j, bf):
            pltpu.make_async_copy(
                rows_v.at[bf], out_hbm.at[batch0 + j], wsem[bf]
            ).wait()

        for bf in range(NBUF):
            start_gather(bf, bf)

        def body(r, carry):
            j0 = r * NBUF
            for bf in range(NBUF):
                wait_gather(j0 + bf, bf)
                start_write(j0 + bf, bf)
            for bf in range(NBUF):
                wait_write(j0 + bf, bf)
                start_gather(j0 + NBUF + bf, bf)
            return carry

        lax.fori_loop(0, rounds - 1, body, 0)

        j0 = (rounds - 1) * NBUF
        for bf in range(NBUF):
            wait_gather(j0 + bf, bf)
            start_write(j0 + bf, bf)
        for bf in range(NBUF):
            wait_write(j0 + bf, bf)

    return k(idx2_flat, pe)


def kernel(timestamps, pe):
    b, h = timestamps.shape
    d = pe.shape[1]
    hp = (h + 7) // 8 * 8
    ts = jnp.clip(timestamps, 0, pe.shape[0] - 1)
    idx2 = jnp.pad(ts, ((0, 0), (0, hp - h)), mode="edge")
    return _sc_gather(idx2.reshape(-1), pe, b, h, hp, d)
